# baseline (speedup 1.0000x reference)
import functools

import jax
import jax.numpy as jnp
from jax import lax
from jax.experimental import pallas as pl

D = 128
_TE = 2000
_TN = 2000
_TA = 200


def _mm(a, b):
    return jnp.dot(a, b, preferred_element_type=jnp.float32)


def _gmat(c, groups):
    s = c // groups
    r = lax.broadcasted_iota(jnp.int32, (c, groups), 0) // s
    g = lax.broadcasted_iota(jnp.int32, (c, groups), 1)
    return (r == g).astype(jnp.float32)


def _gn(x, groups, eps=1e-5):
    c = x.shape[-1]
    s = c // groups
    gm = _gmat(c, groups)
    mean = _mm(_mm(x, gm), gm.T) / s
    xm = x - mean
    var = _mm(_mm(xm * xm, gm), gm.T) / s
    return xm * lax.rsqrt(var + eps)


def _block_diag(w, groups):
    cout = w.shape[0]
    so, si = cout // groups, w.shape[1]
    wg = w.reshape(groups, so, si)
    cin = groups * si
    out = jnp.zeros((cin, cout), jnp.float32)
    for g in range(groups):
        out = out.at[g * si:(g + 1) * si, g * so:(g + 1) * so].set(wg[g].T)
    return out


def _const_spec(x):
    return pl.BlockSpec(x.shape, lambda *_: (0,) * x.ndim)


def _edge_embed_body(attr_ref, tabs_ref, out_ref):
    attr = attr_ref[0]
    oh = jnp.concatenate(
        [(attr[:, f:f + 1] == lax.broadcasted_iota(jnp.int32, (1, 8), 1))
         .astype(jnp.float32) for f in range(3)], axis=1)
    out_ref[...] = _mm(oh, tabs_ref[...])


def _edge_embed(attr3, tabs):
    nt = attr3.shape[0]
    return pl.pallas_call(
        _edge_embed_body,
        grid=(nt,),
        in_specs=[pl.BlockSpec((1, _TE, 3), lambda i: (i, 0, 0)),
                  _const_spec(tabs)],
        out_specs=pl.BlockSpec((_TE, D), lambda i: (i, 0)),
        out_shape=jax.ShapeDtypeStruct((nt * _TE, D), jnp.float32),
    )(attr3, tabs)


def _edge_msg_body(xpd_ref, xps_ref, attr_ref, tabs_ref, wg_ref, wv_ref,
                   wpT_ref, pb_ref, out_ref):
    attr = attr_ref[0]
    oh = jnp.concatenate(
        [(attr[:, f:f + 1] == lax.broadcasted_iota(jnp.int32, (1, 8), 1))
         .astype(jnp.float32) for f in range(3)], axis=1)
    bond = _mm(oh, tabs_ref[...])
    g = _gn(xpd_ref[...].astype(jnp.float32) + bond, 16)
    v = _gn(xps_ref[...].astype(jnp.float32) + bond, 16)
    gate = jnp.maximum(_mm(g, wg_ref[...]), 0.0)
    out_ref[...] = _mm(gate * _mm(v, wv_ref[...]), wpT_ref[...]) + pb_ref[...]


def _edge_msg(xpcat, attr3, tabs, wg, wv, wpT, pb):
    nt = attr3.shape[0]
    return pl.pallas_call(
        _edge_msg_body,
        grid=(nt,),
        in_specs=[pl.BlockSpec((_TE, D), lambda i: (i, 0)),
                  pl.BlockSpec((_TE, D), lambda i, nt=nt: (i + nt, 0)),
                  pl.BlockSpec((1, _TE, 3), lambda i: (i, 0, 0)),
                  _const_spec(tabs), _const_spec(wg), _const_spec(wv),
                  _const_spec(wpT), _const_spec(pb)],
        out_specs=pl.BlockSpec((_TE, D), lambda i: (i, 0)),
        out_shape=jax.ShapeDtypeStruct((nt * _TE, D), jnp.float32),
    )(xpcat, xpcat, attr3, tabs, wg, wv, wpT, pb)


def _atom_body(x_ref, tabs_ref, out_ref):
    x = x_ref[0]
    oh = jnp.concatenate(
        [(x[:, f:f + 1] == lax.broadcasted_iota(jnp.int32, (1, 128), 1))
         .astype(jnp.float32) for f in range(9)], axis=1)
    out_ref[...] = _mm(oh, tabs_ref[...])


def _atom_embed(x3, tabs):
    nt = x3.shape[0]
    return pl.pallas_call(
        _atom_body,
        grid=(nt,),
        in_specs=[pl.BlockSpec((1, _TN, 9), lambda i: (i, 0, 0)),
                  _const_spec(tabs)],
        out_specs=pl.BlockSpec((_TN, D), lambda i: (i, 0)),
        out_shape=jax.ShapeDtypeStruct((nt * _TN, D), jnp.float32),
    )(x3, tabs)


def _scale_add_body(base_ref, acc_ref, nd_ref, sc_ref, out_ref):
    nd = nd_ref[0, 0, :][:, None]
    oh = (nd == lax.broadcasted_iota(jnp.int32, (1, 4), 1)).astype(jnp.float32)
    out_ref[...] = base_ref[...] + _mm(oh, jnp.exp(sc_ref[...])) * acc_ref[...]


def _scale_add(base, acc, nd3, sc):
    nt = nd3.shape[0]
    return pl.pallas_call(
        _scale_add_body,
        grid=(nt,),
        in_specs=[pl.BlockSpec((_TN, D), lambda i: (i, 0)),
                  pl.BlockSpec((_TN, D), lambda i: (i, 0)),
                  pl.BlockSpec((1, 1, _TN), lambda i: (i, 0, 0)),
                  _const_spec(sc)],
        out_specs=pl.BlockSpec((_TN, D), lambda i: (i, 0)),
        out_shape=jax.ShapeDtypeStruct(base.shape, jnp.float32),
    )(base, acc, nd3, sc)


def _lin_body(x_ref, wT_ref, out_ref):
    out_ref[...] = _mm(x_ref[...], wT_ref[...])


def _lin(x, wT):
    nt = x.shape[0] // _TN
    return pl.pallas_call(
        _lin_body,
        grid=(nt,),
        in_specs=[pl.BlockSpec((_TN, D), lambda i: (i, 0)), _const_spec(wT)],
        out_specs=pl.BlockSpec((_TN, wT.shape[1]), lambda i: (i, 0)),
        out_shape=jax.ShapeDtypeStruct((x.shape[0], wT.shape[1]), jnp.float32),
    )(x, wT)


def _lin2_body(a_ref, b_ref, wT_ref, out_ref):
    out_ref[...] = _mm(a_ref[...] + b_ref[...], wT_ref[...])


def _lin2(a, b, wT):
    nt = a.shape[0] // _TN
    return pl.pallas_call(
        _lin2_body,
        grid=(nt,),
        in_specs=[pl.BlockSpec((_TN, D), lambda i: (i, 0)),
                  pl.BlockSpec((_TN, D), lambda i: (i, 0)), _const_spec(wT)],
        out_specs=pl.BlockSpec((_TN, wT.shape[1]), lambda i: (i, 0)),
        out_shape=jax.ShapeDtypeStruct((a.shape[0], wT.shape[1]), jnp.float32),
    )(a, b, wT)


def _glb_body(h_ref, preT_ref, preb_ref, wg_ref, wv_ref, wpT_ref, pb_ref,
              out_ref):
    xx = _gn(_mm(h_ref[...], preT_ref[...]) + preb_ref[...], 16)
    gate = jnp.maximum(_mm(xx, wg_ref[...]), 0.0)
    out_ref[...] = _mm(gate * _mm(xx, wv_ref[...]), wpT_ref[...]) + pb_ref[...]


def _glb(h, preT, preb, wg, wv, wpT, pb, tile):
    nt = h.shape[0] // tile
    return pl.pallas_call(
        _glb_body,
        grid=(nt,),
        in_specs=[pl.BlockSpec((tile, D), lambda i: (i, 0)),
                  _const_spec(preT), _const_spec(preb), _const_spec(wg),
                  _const_spec(wv), _const_spec(wpT), _const_spec(pb)],
        out_specs=pl.BlockSpec((tile, D), lambda i: (i, 0)),
        out_shape=jax.ShapeDtypeStruct((h.shape[0], D), jnp.float32),
    )(h, preT, preb, wg, wv, wpT, pb)


def _att_pre_body(h_ref, preT_ref, preb_ref, wq_ref, wk_ref, wv_ref,
                  q_ref, k_ref, v_ref):
    xx = _gn(_mm(h_ref[...], preT_ref[...]) + preb_ref[...], 8)
    q_ref[...] = jnp.exp(_mm(xx, wq_ref[...]) * 0.25)
    k_ref[...] = jnp.exp(_mm(xx, wk_ref[...]) * 0.25)
    v_ref[...] = _mm(xx, wv_ref[...])


def _att_pre(h, preT, preb, wq, wk, wv):
    n = h.shape[0]
    nt = n // _TN
    sh = jax.ShapeDtypeStruct((n, 2 * D), jnp.float32)
    return pl.pallas_call(
        _att_pre_body,
        grid=(nt,),
        in_specs=[pl.BlockSpec((_TN, D), lambda i: (i, 0)),
                  _const_spec(preT), _const_spec(preb), _const_spec(wq),
                  _const_spec(wk), _const_spec(wv)],
        out_specs=[pl.BlockSpec((_TN, 2 * D), lambda i: (i, 0))] * 3,
        out_shape=[sh, sh, sh],
    )(h, preT, preb, wq, wk, wv)


def _seg_sum_body(x_ref, b_ref, prev_ref, out_ref):
    @pl.when(pl.program_id(0) == 0)
    def _():
        out_ref[...] = prev_ref[...]
    b = b_ref[0, 0, :][None, :]
    oh = (lax.broadcasted_iota(jnp.int32, (64, 1), 0) == b).astype(jnp.float32)
    out_ref[...] += _mm(oh, x_ref[...])


def _seg_sum(x, b3, prev):
    nt = b3.shape[0]
    f = x.shape[1]
    return pl.pallas_call(
        _seg_sum_body,
        grid=(nt,),
        in_specs=[pl.BlockSpec((_TA, f), lambda i: (i, 0)),
                  pl.BlockSpec((1, 1, _TA), lambda i: (i, 0, 0)),
                  _const_spec(prev)],
        out_specs=pl.BlockSpec((64, f), lambda i: (0, 0)),
        out_shape=jax.ShapeDtypeStruct((64, f), jnp.float32),
    )(x, b3, prev)


def _seg_kv_body(xk_ref, xv_ref, b_ref, prev_ref, out_ref):
    @pl.when(pl.program_id(0) == 0)
    def _():
        out_ref[...] = prev_ref[...]
    xk = xk_ref[...].reshape(_TA, 16, 16)
    xv = xv_ref[...].reshape(_TA, 16, 16)
    kv = (xk[:, :, :, None] * xv[:, :, None, :]).reshape(_TA, 4096)
    b = b_ref[0, 0, :][None, :]
    oh = (lax.broadcasted_iota(jnp.int32, (64, 1), 0) == b).astype(jnp.float32)
    out_ref[...] += _mm(oh, kv)


def _seg_kv(xk, xv, b3, prev):
    nt = b3.shape[0]
    return pl.pallas_call(
        _seg_kv_body,
        grid=(nt,),
        in_specs=[pl.BlockSpec((_TA, 2 * D), lambda i: (i, 0)),
                  pl.BlockSpec((_TA, 2 * D), lambda i: (i, 0)),
                  pl.BlockSpec((1, 1, _TA), lambda i: (i, 0, 0)),
                  _const_spec(prev)],
        out_specs=pl.BlockSpec((64, 4096), lambda i: (0, 0)),
        out_shape=jax.ShapeDtypeStruct((64, 4096), jnp.float32),
    )(xk, xv, b3, prev)


def _att_out_body(xq_ref, ks_ref, ha_ref, b_ref, b1_ref, b2_ref, vm_ref,
                  vs_ref, postT_ref, pb_ref, as_ref, out_ref):
    b = b_ref[0, 0, :][:, None]
    oh = (b == lax.broadcasted_iota(jnp.int32, (1, 64), 1)).astype(jnp.float32)
    xq = xq_ref[...]
    ks = _mm(oh, ks_ref[...])
    gm = _gmat(2 * D, 16)
    denom = _mm(_mm(xq * ks, gm), gm.T)
    xqn = xq / denom
    ha = _mm(oh, ha_ref[...])
    x4 = xqn.reshape(_TA, 16, 16, 1)
    h4 = ha.reshape(_TA, 16, 16, 16)
    att = jnp.sum(x4 * h4, axis=2).reshape(_TA, 2 * D)
    virt = _mm(oh, vm_ref[...] * jnp.exp(vs_ref[...]))
    post = _mm(att, postT_ref[...]) + pb_ref[...]
    out_ref[...] = (b1_ref[...] + b2_ref[...] + virt
                    + jnp.exp(as_ref[...]) * post)


def _att_out(xq, ksum, hatt, b3, b1, b2, vm, vs, postT, pb, ascale):
    nt = b3.shape[0]
    return pl.pallas_call(
        _att_out_body,
        grid=(nt,),
        in_specs=[pl.BlockSpec((_TA, 2 * D), lambda i: (i, 0)),
                  _const_spec(ksum), _const_spec(hatt),
                  pl.BlockSpec((1, 1, _TA), lambda i: (i, 0, 0)),
                  pl.BlockSpec((_TA, D), lambda i: (i, 0)),
                  pl.BlockSpec((_TA, D), lambda i: (i, 0)),
                  _const_spec(vm), _const_spec(vs), _const_spec(postT),
                  _const_spec(pb), _const_spec(ascale)],
        out_specs=pl.BlockSpec((_TA, D), lambda i: (i, 0)),
        out_shape=jax.ShapeDtypeStruct((b1.shape[0], D), jnp.float32),
    )(xq, ksum, hatt, b3, b1, b2, vm, vs, postT, pb, ascale)


def _gather_rows(table, idx):
    return table[idx]


def _scatter_add(rows, idx, n):
    return jax.ops.segment_sum(rows, idx, num_segments=n)


@jax.jit
def kernel(params, x, edge_index, edge_attr, batch, ptr, distance_bin):
    n = x.shape[0]
    e = edge_index.shape[1]
    src = edge_index[0].astype(jnp.int32)
    dst = edge_index[1].astype(jnp.int32)

    attr3 = edge_attr.astype(jnp.int32).reshape(e // _TE, _TE, 3)
    ds_idx = jnp.concatenate([dst, src])
    x3 = x.astype(jnp.int32).reshape(n // _TN, _TN, 9)
    b3a = batch.astype(jnp.int32).reshape(n // _TA, 1, _TA)
    zeros_n = jnp.zeros((n, D), jnp.float32)

    deg = _scatter_add(jnp.ones((e, 1), jnp.float32), dst, n)[:, 0]
    nd = jnp.clip(deg.astype(jnp.int32) - 1, 0, 3)
    nd3 = nd.reshape(n // _TN, 1, _TN)

    bond0 = _edge_embed(attr3, params['bond_emb0'].reshape(24, D))
    bond0n = _scatter_add(bond0, dst, n)
    atom = _atom_embed(x3, params['atom_emb'].reshape(9 * 128, D))
    h_in = _scale_add(atom, bond0n, nd3, params['scale0'])

    h_att = jnp.zeros((64, 4096), jnp.float32)
    h_virt = jnp.zeros((64, D), jnp.float32)

    for lp in params['layers']:
        x_out = zeros_n
        x_raw = h_in
        for k, cp in enumerate(lp['conv']):
            if k == 0:
                xp = _lin(h_in, cp['pre_w'].T)
            elif k == 2:
                xp = _lin2(h_in, x_out, cp['pre_w'].T)
                x_out = zeros_n
            else:
                xp = _lin(x_raw, cp['pre_w'].T)
            mp = cp['msg']
            msg = _edge_msg(
                xp.astype(jnp.bfloat16)[ds_idx], attr3,
                cp['bond_emb'].reshape(24, D),
                _block_diag(mp['gate_w'], 16), _block_diag(mp['value_w'], 16),
                mp['post_w'].T, mp['post_b'][None, :])
            x_raw = _scatter_add(msg, dst, n)
            x_out = _scale_add(x_out, x_raw, nd3, cp['scale'])

        vp = lp['virt']
        h_virt = _seg_sum(h_in, b3a, h_virt)
        vm = vp['msg']
        vmsg = _glb(h_virt, vm['pre_w'].T, vm['pre_b'][None, :],
                    _block_diag(vm['gate_w'], 16), _block_diag(vm['value_w'], 16),
                    vm['post_w'].T, vm['post_b'][None, :], 64)

        ap = lp['att']
        xq, xk, xv = _att_pre(h_in, ap['pre_w'].T, ap['pre_b'][None, :],
                              _block_diag(ap['msgq_w'], 8),
                              _block_diag(ap['msgk_w'], 8),
                              _block_diag(ap['msgv_w'], 8))
        h_att = _seg_kv(xk, xv, b3a, h_att)
        ksum = _seg_sum(xk, b3a, jnp.zeros((64, 2 * D), jnp.float32))
        h_out = _att_out(xq, ksum, h_att, b3a, h_in, x_out, vmsg,
                         vp['scale'][None, :], ap['post_w'].T,
                         ap['post_b'][None, :], ap['scale'][None, :])

        mn = lp['main']
        h_in = _glb(h_out, mn['pre_w'].T, mn['pre_b'][None, :],
                    _block_diag(mn['gate_w'], 16), _block_diag(mn['value_w'], 16),
                    mn['post_w'].T, mn['post_b'][None, :], _TN)

    return h_in


# speedup vs baseline: 1.3226x; 1.3226x over previous
import functools

import jax
import jax.numpy as jnp
from jax import lax
from jax.experimental import pallas as pl
from jax.experimental.pallas import tpu as pltpu
from jax.experimental.pallas import tpu_sc as plsc

D = 128
_TE = 2000
_TN = 2000
_TA = 200


def _mm(a, b):
    return jnp.dot(a, b, preferred_element_type=jnp.float32)


def _gmat(c, groups):
    s = c // groups
    r = lax.broadcasted_iota(jnp.int32, (c, groups), 0) // s
    g = lax.broadcasted_iota(jnp.int32, (c, groups), 1)
    return (r == g).astype(jnp.float32)


def _gn(x, groups, eps=1e-5):
    c = x.shape[-1]
    s = c // groups
    gm = _gmat(c, groups)
    mean = _mm(_mm(x, gm), gm.T) / s
    xm = x - mean
    var = _mm(_mm(xm * xm, gm), gm.T) / s
    return xm * lax.rsqrt(var + eps)


def _block_diag(w, groups):
    cout = w.shape[0]
    so, si = cout // groups, w.shape[1]
    wg = w.reshape(groups, so, si)
    cin = groups * si
    out = jnp.zeros((cin, cout), jnp.float32)
    for g in range(groups):
        out = out.at[g * si:(g + 1) * si, g * so:(g + 1) * so].set(wg[g].T)
    return out


def _const_spec(x):
    return pl.BlockSpec(x.shape, lambda *_: (0,) * x.ndim)


def _edge_embed_body(attr_ref, tabs_ref, out_ref):
    attr = attr_ref[0]
    oh = jnp.concatenate(
        [(attr[:, f:f + 1] == lax.broadcasted_iota(jnp.int32, (1, 8), 1))
         .astype(jnp.float32) for f in range(3)], axis=1)
    out_ref[...] = _mm(oh, tabs_ref[...])


def _edge_embed(attr3, tabs):
    nt = attr3.shape[0]
    return pl.pallas_call(
        _edge_embed_body,
        grid=(nt,),
        in_specs=[pl.BlockSpec((1, _TE, 3), lambda i: (i, 0, 0)),
                  _const_spec(tabs)],
        out_specs=pl.BlockSpec((_TE, D), lambda i: (i, 0)),
        out_shape=jax.ShapeDtypeStruct((nt * _TE, D), jnp.float32),
    )(attr3, tabs)


def _edge_msg_body(xpd_ref, xps_ref, attr_ref, tabs_ref, wg_ref, wv_ref,
                   wpT_ref, pb_ref, out_ref):
    attr = attr_ref[0]
    oh = jnp.concatenate(
        [(attr[:, f:f + 1] == lax.broadcasted_iota(jnp.int32, (1, 8), 1))
         .astype(jnp.float32) for f in range(3)], axis=1)
    bond = _mm(oh, tabs_ref[...])
    g = _gn(xpd_ref[...].astype(jnp.float32) + bond, 16)
    v = _gn(xps_ref[...].astype(jnp.float32) + bond, 16)
    gate = jnp.maximum(_mm(g, wg_ref[...]), 0.0)
    out_ref[...] = _mm(gate * _mm(v, wv_ref[...]), wpT_ref[...]) + pb_ref[...]


def _edge_msg(xpcat, attr3, tabs, wg, wv, wpT, pb):
    nt = attr3.shape[0]
    return pl.pallas_call(
        _edge_msg_body,
        grid=(nt,),
        in_specs=[pl.BlockSpec((_TE, D), lambda i: (i, 0)),
                  pl.BlockSpec((_TE, D), lambda i, nt=nt: (i + nt, 0)),
                  pl.BlockSpec((1, _TE, 3), lambda i: (i, 0, 0)),
                  _const_spec(tabs), _const_spec(wg), _const_spec(wv),
                  _const_spec(wpT), _const_spec(pb)],
        out_specs=pl.BlockSpec((_TE, D), lambda i: (i, 0)),
        out_shape=jax.ShapeDtypeStruct((nt * _TE, D), jnp.float32),
    )(xpcat, xpcat, attr3, tabs, wg, wv, wpT, pb)


def _atom_body(x_ref, tabs_ref, out_ref):
    x = x_ref[0]
    oh = jnp.concatenate(
        [(x[:, f:f + 1] == lax.broadcasted_iota(jnp.int32, (1, 128), 1))
         .astype(jnp.float32) for f in range(9)], axis=1)
    out_ref[...] = _mm(oh, tabs_ref[...])


def _atom_embed(x3, tabs):
    nt = x3.shape[0]
    return pl.pallas_call(
        _atom_body,
        grid=(nt,),
        in_specs=[pl.BlockSpec((1, _TN, 9), lambda i: (i, 0, 0)),
                  _const_spec(tabs)],
        out_specs=pl.BlockSpec((_TN, D), lambda i: (i, 0)),
        out_shape=jax.ShapeDtypeStruct((nt * _TN, D), jnp.float32),
    )(x3, tabs)


def _scale_add_body(base_ref, acc_ref, nd_ref, sc_ref, out_ref):
    nd = nd_ref[0, 0, :][:, None]
    oh = (nd == lax.broadcasted_iota(jnp.int32, (1, 4), 1)).astype(jnp.float32)
    out_ref[...] = base_ref[...] + _mm(oh, jnp.exp(sc_ref[...])) * acc_ref[...]


def _scale_add(base, acc, nd3, sc):
    nt = nd3.shape[0]
    return pl.pallas_call(
        _scale_add_body,
        grid=(nt,),
        in_specs=[pl.BlockSpec((_TN, D), lambda i: (i, 0)),
                  pl.BlockSpec((_TN, D), lambda i: (i, 0)),
                  pl.BlockSpec((1, 1, _TN), lambda i: (i, 0, 0)),
                  _const_spec(sc)],
        out_specs=pl.BlockSpec((_TN, D), lambda i: (i, 0)),
        out_shape=jax.ShapeDtypeStruct(base.shape, jnp.float32),
    )(base, acc, nd3, sc)


def _lin_body(x_ref, wT_ref, out_ref):
    out_ref[...] = _mm(x_ref[...], wT_ref[...])


def _lin(x, wT):
    nt = x.shape[0] // _TN
    return pl.pallas_call(
        _lin_body,
        grid=(nt,),
        in_specs=[pl.BlockSpec((_TN, D), lambda i: (i, 0)), _const_spec(wT)],
        out_specs=pl.BlockSpec((_TN, wT.shape[1]), lambda i: (i, 0)),
        out_shape=jax.ShapeDtypeStruct((x.shape[0], wT.shape[1]), jnp.float32),
    )(x, wT)


def _lin2_body(a_ref, b_ref, wT_ref, out_ref):
    out_ref[...] = _mm(a_ref[...] + b_ref[...], wT_ref[...])


def _lin2(a, b, wT):
    nt = a.shape[0] // _TN
    return pl.pallas_call(
        _lin2_body,
        grid=(nt,),
        in_specs=[pl.BlockSpec((_TN, D), lambda i: (i, 0)),
                  pl.BlockSpec((_TN, D), lambda i: (i, 0)), _const_spec(wT)],
        out_specs=pl.BlockSpec((_TN, wT.shape[1]), lambda i: (i, 0)),
        out_shape=jax.ShapeDtypeStruct((a.shape[0], wT.shape[1]), jnp.float32),
    )(a, b, wT)


def _glb_body(h_ref, preT_ref, preb_ref, wg_ref, wv_ref, wpT_ref, pb_ref,
              out_ref):
    xx = _gn(_mm(h_ref[...], preT_ref[...]) + preb_ref[...], 16)
    gate = jnp.maximum(_mm(xx, wg_ref[...]), 0.0)
    out_ref[...] = _mm(gate * _mm(xx, wv_ref[...]), wpT_ref[...]) + pb_ref[...]


def _glb(h, preT, preb, wg, wv, wpT, pb, tile):
    nt = h.shape[0] // tile
    return pl.pallas_call(
        _glb_body,
        grid=(nt,),
        in_specs=[pl.BlockSpec((tile, D), lambda i: (i, 0)),
                  _const_spec(preT), _const_spec(preb), _const_spec(wg),
                  _const_spec(wv), _const_spec(wpT), _const_spec(pb)],
        out_specs=pl.BlockSpec((tile, D), lambda i: (i, 0)),
        out_shape=jax.ShapeDtypeStruct((h.shape[0], D), jnp.float32),
    )(h, preT, preb, wg, wv, wpT, pb)


def _att_pre_body(h_ref, preT_ref, preb_ref, wq_ref, wk_ref, wv_ref,
                  q_ref, k_ref, v_ref):
    xx = _gn(_mm(h_ref[...], preT_ref[...]) + preb_ref[...], 8)
    q_ref[...] = jnp.exp(_mm(xx, wq_ref[...]) * 0.25)
    k_ref[...] = jnp.exp(_mm(xx, wk_ref[...]) * 0.25)
    v_ref[...] = _mm(xx, wv_ref[...])


def _att_pre(h, preT, preb, wq, wk, wv):
    n = h.shape[0]
    nt = n // _TN
    sh = jax.ShapeDtypeStruct((n, 2 * D), jnp.float32)
    return pl.pallas_call(
        _att_pre_body,
        grid=(nt,),
        in_specs=[pl.BlockSpec((_TN, D), lambda i: (i, 0)),
                  _const_spec(preT), _const_spec(preb), _const_spec(wq),
                  _const_spec(wk), _const_spec(wv)],
        out_specs=[pl.BlockSpec((_TN, 2 * D), lambda i: (i, 0))] * 3,
        out_shape=[sh, sh, sh],
    )(h, preT, preb, wq, wk, wv)


def _seg_sum_body(x_ref, b_ref, prev_ref, out_ref):
    @pl.when(pl.program_id(0) == 0)
    def _():
        out_ref[...] = prev_ref[...]
    b = b_ref[0, 0, :][None, :]
    oh = (lax.broadcasted_iota(jnp.int32, (64, 1), 0) == b).astype(jnp.float32)
    out_ref[...] += _mm(oh, x_ref[...])


def _seg_sum(x, b3, prev):
    nt = b3.shape[0]
    f = x.shape[1]
    return pl.pallas_call(
        _seg_sum_body,
        grid=(nt,),
        in_specs=[pl.BlockSpec((_TA, f), lambda i: (i, 0)),
                  pl.BlockSpec((1, 1, _TA), lambda i: (i, 0, 0)),
                  _const_spec(prev)],
        out_specs=pl.BlockSpec((64, f), lambda i: (0, 0)),
        out_shape=jax.ShapeDtypeStruct((64, f), jnp.float32),
    )(x, b3, prev)


def _seg_kv_body(xk_ref, xv_ref, b_ref, prev_ref, out_ref):
    @pl.when(pl.program_id(0) == 0)
    def _():
        out_ref[...] = prev_ref[...]
    xk = xk_ref[...].reshape(_TA, 16, 16)
    xv = xv_ref[...].reshape(_TA, 16, 16)
    kv = (xk[:, :, :, None] * xv[:, :, None, :]).reshape(_TA, 4096)
    b = b_ref[0, 0, :][None, :]
    oh = (lax.broadcasted_iota(jnp.int32, (64, 1), 0) == b).astype(jnp.float32)
    out_ref[...] += _mm(oh, kv)


def _seg_kv(xk, xv, b3, prev):
    nt = b3.shape[0]
    return pl.pallas_call(
        _seg_kv_body,
        grid=(nt,),
        in_specs=[pl.BlockSpec((_TA, 2 * D), lambda i: (i, 0)),
                  pl.BlockSpec((_TA, 2 * D), lambda i: (i, 0)),
                  pl.BlockSpec((1, 1, _TA), lambda i: (i, 0, 0)),
                  _const_spec(prev)],
        out_specs=pl.BlockSpec((64, 4096), lambda i: (0, 0)),
        out_shape=jax.ShapeDtypeStruct((64, 4096), jnp.float32),
    )(xk, xv, b3, prev)


def _att_out_body(xq_ref, ks_ref, ha_ref, b_ref, b1_ref, b2_ref, vm_ref,
                  vs_ref, postT_ref, pb_ref, as_ref, out_ref):
    b = b_ref[0, 0, :][:, None]
    oh = (b == lax.broadcasted_iota(jnp.int32, (1, 64), 1)).astype(jnp.float32)
    xq = xq_ref[...]
    ks = _mm(oh, ks_ref[...])
    gm = _gmat(2 * D, 16)
    denom = _mm(_mm(xq * ks, gm), gm.T)
    xqn = xq / denom
    ha = _mm(oh, ha_ref[...])
    x4 = xqn.reshape(_TA, 16, 16, 1)
    h4 = ha.reshape(_TA, 16, 16, 16)
    att = jnp.sum(x4 * h4, axis=2).reshape(_TA, 2 * D)
    virt = _mm(oh, vm_ref[...] * jnp.exp(vs_ref[...]))
    post = _mm(att, postT_ref[...]) + pb_ref[...]
    out_ref[...] = (b1_ref[...] + b2_ref[...] + virt
                    + jnp.exp(as_ref[...]) * post)


def _att_out(xq, ksum, hatt, b3, b1, b2, vm, vs, postT, pb, ascale):
    nt = b3.shape[0]
    return pl.pallas_call(
        _att_out_body,
        grid=(nt,),
        in_specs=[pl.BlockSpec((_TA, 2 * D), lambda i: (i, 0)),
                  _const_spec(ksum), _const_spec(hatt),
                  pl.BlockSpec((1, 1, _TA), lambda i: (i, 0, 0)),
                  pl.BlockSpec((_TA, D), lambda i: (i, 0)),
                  pl.BlockSpec((_TA, D), lambda i: (i, 0)),
                  _const_spec(vm), _const_spec(vs), _const_spec(postT),
                  _const_spec(pb), _const_spec(ascale)],
        out_specs=pl.BlockSpec((_TA, D), lambda i: (i, 0)),
        out_shape=jax.ShapeDtypeStruct((b1.shape[0], D), jnp.float32),
    )(xq, ksum, hatt, b3, b1, b2, vm, vs, postT, pb, ascale)


def _sc_gather(table, idx):
    b = idx.shape[0]
    nw = 32
    bw = b // nw
    ch = 400
    mesh = plsc.VectorSubcoreMesh(core_axis_name="c", subcore_axis_name="s")

    @functools.partial(
        pl.kernel, mesh=mesh,
        out_type=jax.ShapeDtypeStruct((b, D), jnp.float32),
        scratch_types=[pltpu.VMEM((ch,), jnp.int32),
                       pltpu.VMEM((ch, D), jnp.float32),
                       pltpu.SemaphoreType.DMA],
    )
    def k(table_hbm, idx_hbm, out_hbm, idx_v, rows_v, sem):
        wid = lax.axis_index("s") * 2 + lax.axis_index("c")

        def body(c, carry):
            off = wid * bw + c * ch
            pltpu.sync_copy(idx_hbm.at[pl.ds(off, ch)], idx_v)
            pltpu.async_copy(table_hbm.at[idx_v], rows_v, sem).wait()
            pltpu.sync_copy(rows_v, out_hbm.at[pl.ds(off, ch)])
            return carry

        lax.fori_loop(0, bw // ch, body, 0)

    return k(table, idx)


def _scatter_add(rows, idx, n):
    return jax.ops.segment_sum(rows, idx, num_segments=n)


@jax.jit
def kernel(params, x, edge_index, edge_attr, batch, ptr, distance_bin):
    n = x.shape[0]
    e = edge_index.shape[1]
    src = edge_index[0].astype(jnp.int32)
    dst = edge_index[1].astype(jnp.int32)

    attr3 = edge_attr.astype(jnp.int32).reshape(e // _TE, _TE, 3)
    ds_idx = jnp.concatenate([dst, src])
    x3 = x.astype(jnp.int32).reshape(n // _TN, _TN, 9)
    b3a = batch.astype(jnp.int32).reshape(n // _TA, 1, _TA)
    zeros_n = jnp.zeros((n, D), jnp.float32)

    deg = _scatter_add(jnp.ones((e, 1), jnp.float32), dst, n)[:, 0]
    nd = jnp.clip(deg.astype(jnp.int32) - 1, 0, 3)
    nd3 = nd.reshape(n // _TN, 1, _TN)

    bond0 = _edge_embed(attr3, params['bond_emb0'].reshape(24, D))
    bond0n = _scatter_add(bond0, dst, n)
    atom = _atom_embed(x3, params['atom_emb'].reshape(9 * 128, D))
    h_in = _scale_add(atom, bond0n, nd3, params['scale0'])

    h_att = jnp.zeros((64, 4096), jnp.float32)
    h_virt = jnp.zeros((64, D), jnp.float32)

    for lp in params['layers']:
        x_out = zeros_n
        x_raw = h_in
        for k, cp in enumerate(lp['conv']):
            if k == 0:
                xp = _lin(h_in, cp['pre_w'].T)
            elif k == 2:
                xp = _lin2(h_in, x_out, cp['pre_w'].T)
                x_out = zeros_n
            else:
                xp = _lin(x_raw, cp['pre_w'].T)
            mp = cp['msg']
            msg = _edge_msg(
                _sc_gather(xp, ds_idx), attr3,
                cp['bond_emb'].reshape(24, D),
                _block_diag(mp['gate_w'], 16), _block_diag(mp['value_w'], 16),
                mp['post_w'].T, mp['post_b'][None, :])
            x_raw = _scatter_add(msg, dst, n)
            x_out = _scale_add(x_out, x_raw, nd3, cp['scale'])

        vp = lp['virt']
        h_virt = _seg_sum(h_in, b3a, h_virt)
        vm = vp['msg']
        vmsg = _glb(h_virt, vm['pre_w'].T, vm['pre_b'][None, :],
                    _block_diag(vm['gate_w'], 16), _block_diag(vm['value_w'], 16),
                    vm['post_w'].T, vm['post_b'][None, :], 64)

        ap = lp['att']
        xq, xk, xv = _att_pre(h_in, ap['pre_w'].T, ap['pre_b'][None, :],
                              _block_diag(ap['msgq_w'], 8),
                              _block_diag(ap['msgk_w'], 8),
                              _block_diag(ap['msgv_w'], 8))
        h_att = _seg_kv(xk, xv, b3a, h_att)
        ksum = _seg_sum(xk, b3a, jnp.zeros((64, 2 * D), jnp.float32))
        h_out = _att_out(xq, ksum, h_att, b3a, h_in, x_out, vmsg,
                         vp['scale'][None, :], ap['post_w'].T,
                         ap['post_b'][None, :], ap['scale'][None, :])

        mn = lp['main']
        h_in = _glb(h_out, mn['pre_w'].T, mn['pre_b'][None, :],
                    _block_diag(mn['gate_w'], 16), _block_diag(mn['value_w'], 16),
                    mn['post_w'].T, mn['post_b'][None, :], _TN)

    return h_in
